# baseline (speedup 1.0000x reference)
import functools

import jax
import jax.numpy as jnp
from jax import lax
from jax.experimental import pallas as pl
from jax.experimental.pallas import tpu as pltpu
from jax.experimental.pallas import tpu_sc as plsc

_NUM_WORKERS = 32
_LANES = 16
_NBUF = 3
_IDXSTAGE = 136


def kernel(x, token_emb, pos_emb):
    B, S = x.shape
    V, D = token_emb.shape
    HALF = S // 2
    NBLK = B // _NUM_WORKERS
    SPLIT = _IDXSTAGE // 2

    x2 = x.reshape(B * 2, HALF).astype(jnp.int32)

    mesh = plsc.VectorSubcoreMesh(core_axis_name="c", subcore_axis_name="s")

    @functools.partial(
        pl.kernel,
        mesh=mesh,
        out_type=jax.ShapeDtypeStruct((B * S, D), jnp.float32),
        scratch_types=[
            pltpu.VMEM((_IDXSTAGE, HALF), jnp.int32),
            pltpu.VMEM((S, D), jnp.float32),
        ] + [pltpu.VMEM((S, D), jnp.float32) for _ in range(_NBUF)]
          + [pltpu.SemaphoreType.DMA for _ in range(2 * _NBUF + 1)],
    )
    def run(tok_hbm, idx_hbm, pos_hbm, out_hbm, idx_v, pos_v, *rest):
        bufs = rest[:_NBUF]
        gsems = rest[_NBUF:2 * _NBUF]
        wsems = rest[2 * _NBUF:3 * _NBUF]
        rsem = rest[3 * _NBUF]
        wid = lax.axis_index("s") * 2 + lax.axis_index("c")
        seq_base = wid * NBLK
        pltpu.sync_copy(idx_hbm.at[pl.ds(seq_base * 2, _IDXSTAGE)], idx_v)
        pltpu.sync_copy(pos_hbm, pos_v)

        RELOAD_ROWS = 2 * NBLK - _IDXSTAGE

        def reload_copy():
            return pltpu.make_async_copy(
                idx_hbm.at[pl.ds(seq_base * 2 + _IDXSTAGE, RELOAD_ROWS)],
                idx_v.at[pl.ds(0, RELOAD_ROWS)], rsem)

        def idx_row(blk):
            return 2 * blk - jnp.where(blk >= SPLIT, _IDXSTAGE, 0)

        def issue_gather(blk, j):
            off = idx_row(blk)
            pltpu.async_copy(tok_hbm.at[idx_v.at[off]],
                             bufs[j].at[pl.ds(0, HALF)], gsems[j])
            pltpu.async_copy(tok_hbm.at[idx_v.at[off + 1]],
                             bufs[j].at[pl.ds(HALF, HALF)], gsems[j])

        def wait_gather(blk, j):
            off = idx_row(blk)
            pltpu.make_async_copy(tok_hbm.at[idx_v.at[off]],
                                  bufs[j].at[pl.ds(0, HALF)], gsems[j]).wait()
            pltpu.make_async_copy(tok_hbm.at[idx_v.at[off + 1]],
                                  bufs[j].at[pl.ds(HALF, HALF)],
                                  gsems[j]).wait()

        def issue_writeback(blk, j):
            pltpu.async_copy(bufs[j],
                             out_hbm.at[pl.ds((seq_base + blk) * S, S)],
                             wsems[j])

        def wait_writeback(j):
            pltpu.make_async_copy(bufs[j], out_hbm.at[pl.ds(0, S)],
                                  wsems[j]).wait()

        def add_pos(j):
            buf = bufs[j]

            @pl.loop(0, S, step=8)
            def _(r):
                for dr in range(8):
                    for c in range(D // _LANES):
                        sl = pl.ds(c * _LANES, _LANES)
                        plsc.addupdate(buf.at[r + dr, sl], pos_v[r + dr, sl])

        for j in range(_NBUF - 1):
            issue_gather(j, j)

        NT = (NBLK + _NBUF - 1) // _NBUF

        @pl.loop(0, NT)
        def _(t):
            for b in range(_NBUF):
                blk = _NBUF * t + b
                jg = (b + _NBUF - 1) % _NBUF
                blk_g = blk + _NBUF - 1

                @pl.when(blk == SPLIT - 8)
                def _():
                    reload_copy().start()

                @pl.when(blk_g < NBLK)
                def _():
                    @pl.when(blk >= 1)
                    def _():
                        wait_writeback(jg)

                    @pl.when(blk_g == SPLIT)
                    def _():
                        reload_copy().wait()
                    issue_gather(blk_g, jg)

                @pl.when(blk < NBLK)
                def _():
                    wait_gather(blk, b)
                    add_pos(b)
                    issue_writeback(blk, b)

        for j in range(_NBUF):
            wait_writeback(j)

    out = run(token_emb, x2, pos_emb)
    return out.reshape(B, S, D)
